# baseline (speedup 1.0000x reference)
import functools

import jax
import jax.numpy as jnp
from jax import lax
from jax.experimental import pallas as pl
from jax.experimental.pallas import tpu as pltpu
from jax.experimental.pallas import tpu_sc as plsc

_B = 16384
_D = 64
_CHUNK = 128


def _make_gather():
    info = plsc.get_sparse_core_info()
    num_cores = info.num_cores
    nw = num_cores * info.num_subcores
    b_per_w = _B // nw
    n_chunks = b_per_w // _CHUNK
    mesh = plsc.VectorSubcoreMesh(core_axis_name="c", subcore_axis_name="s")

    @functools.partial(
        pl.kernel,
        mesh=mesh,
        out_type=jax.ShapeDtypeStruct((_B, _D), jnp.float32),
        scratch_types=[
            pltpu.VMEM((b_per_w,), jnp.int32),
            pltpu.VMEM((b_per_w, _D), jnp.float32),
            pltpu.SemaphoreType.DMA,
        ],
        compiler_params=pltpu.CompilerParams(use_tc_tiling_on_sc=False),
    )
    def gather_kernel(emb_hbm, idx_hbm, out_hbm, idx_v, rows_v, sem):
        wid = lax.axis_index("s") * num_cores + lax.axis_index("c")
        base = wid * b_per_w
        pltpu.sync_copy(idx_hbm.at[pl.ds(base, b_per_w)], idx_v)
        copies = [
            pltpu.async_copy(
                emb_hbm.at[idx_v.at[pl.ds(j * _CHUNK, _CHUNK)]],
                rows_v.at[pl.ds(j * _CHUNK, _CHUNK)],
                sem,
            )
            for j in range(n_chunks)
        ]
        for c in copies:
            c.wait()
        pltpu.sync_copy(rows_v, out_hbm.at[pl.ds(base, b_per_w)])

    return gather_kernel


_gather = _make_gather()


def kernel(emb, token_index):
    return _gather(emb, token_index.astype(jnp.int32))


# speedup vs baseline: 1.4986x; 1.4986x over previous
import functools

import jax
import jax.numpy as jnp
from jax import lax
from jax.experimental import pallas as pl
from jax.experimental.pallas import tpu as pltpu
from jax.experimental.pallas import tpu_sc as plsc

_B = 16384
_D = 64


def _make_gather():
    info = plsc.get_sparse_core_info()
    num_cores = info.num_cores
    nw = num_cores * info.num_subcores
    b_per_w = _B // nw
    mesh = plsc.VectorSubcoreMesh(core_axis_name="c", subcore_axis_name="s")

    @functools.partial(
        pl.kernel,
        mesh=mesh,
        out_type=jax.ShapeDtypeStruct((_B, _D), jnp.float32),
        scratch_types=[
            pltpu.VMEM((b_per_w,), jnp.int32),
            pltpu.VMEM((b_per_w, _D), jnp.float32),
            pltpu.SemaphoreType.DMA,
        ],
    )
    def gather_kernel(emb_hbm, idx_hbm, out_hbm, idx_v, rows_v, sem):
        wid = lax.axis_index("s") * num_cores + lax.axis_index("c")
        base = wid * b_per_w
        pltpu.sync_copy(idx_hbm.at[pl.ds(base, b_per_w)], idx_v)

        def body(j, carry):
            vec = idx_v[pl.ds(j * 16, 16)]
            for lane in range(16):
                r = vec[lane]
                pltpu.async_copy(
                    emb_hbm.at[pl.ds(r, 1)],
                    rows_v.at[pl.ds(j * 16 + lane, 1)],
                    sem,
                )
            return carry

        lax.fori_loop(0, b_per_w // 16, body, 0)
        pltpu.make_async_copy(
            emb_hbm.at[pl.ds(0, b_per_w)], rows_v, sem
        ).wait()
        pltpu.sync_copy(rows_v, out_hbm.at[pl.ds(base, b_per_w)])

    return gather_kernel


_gather = _make_gather()


def kernel(emb, token_index):
    return _gather(emb, token_index.astype(jnp.int32))


# speedup vs baseline: 1.8589x; 1.2405x over previous
import functools

import jax
import jax.numpy as jnp
from jax import lax
from jax.experimental import pallas as pl
from jax.experimental.pallas import tpu as pltpu
from jax.experimental.pallas import tpu_sc as plsc

_B = 16384
_D = 64
_V = 100000
_CH = 2048


def _make_gather():
    info = plsc.get_sparse_core_info()
    num_cores = info.num_cores
    nw = num_cores * info.num_subcores
    d_per_w = _D // nw
    mesh = plsc.VectorSubcoreMesh(core_axis_name="c", subcore_axis_name="s")

    @functools.partial(
        pl.kernel,
        mesh=mesh,
        out_type=jax.ShapeDtypeStruct((_D, _B), jnp.float32),
        scratch_types=[
            pltpu.VMEM((_V,), jnp.float32),
            pltpu.VMEM((_B,), jnp.int32),
            pltpu.VMEM((_CH,), jnp.float32),
        ],
        compiler_params=pltpu.CompilerParams(needs_layout_passes=False),
    )
    def gather_kernel(embT_hbm, idx_hbm, outT_hbm, row_v, idx_v, out_c):
        wid = lax.axis_index("s") * num_cores + lax.axis_index("c")
        pltpu.sync_copy(idx_hbm, idx_v)
        for rep in range(d_per_w):
            d = wid + rep * nw
            pltpu.sync_copy(embT_hbm.at[d], row_v)
            for c in range(_B // _CH):

                def body(j, carry, c=c):
                    iv = idx_v[pl.ds(c * _CH + j * 16, 16)]
                    out_c[pl.ds(j * 16, 16)] = plsc.load_gather(row_v, [iv])
                    return carry

                lax.fori_loop(0, _CH // 16, body, 0)
                pltpu.sync_copy(out_c, outT_hbm.at[d, pl.ds(c * _CH, _CH)])

    return gather_kernel


_gather = _make_gather()


def kernel(emb, token_index):
    outT = _gather(emb.T, token_index.astype(jnp.int32))
    return outT.T


# speedup vs baseline: 2.7225x; 1.4646x over previous
import functools

import jax
import jax.numpy as jnp
from jax import lax
from jax.experimental import pallas as pl
from jax.experimental.pallas import tpu as pltpu
from jax.experimental.pallas import tpu_sc as plsc

_B = 16384
_D = 64
_V = 100000
_CH = 2048


def _make_gather():
    info = plsc.get_sparse_core_info()
    num_cores = info.num_cores
    nw = num_cores * info.num_subcores
    d_per_w = _D // nw
    n_chunks = _B // _CH
    mesh = plsc.VectorSubcoreMesh(core_axis_name="c", subcore_axis_name="s")

    @functools.partial(
        pl.kernel,
        mesh=mesh,
        out_type=jax.ShapeDtypeStruct((_D, _B), jnp.float32),
        scratch_types=[
            pltpu.VMEM((_V,), jnp.float32),
            pltpu.VMEM((_B,), jnp.int32),
            pltpu.VMEM((_CH,), jnp.float32),
            pltpu.VMEM((_CH,), jnp.float32),
            pltpu.SemaphoreType.DMA,
            pltpu.SemaphoreType.DMA,
            pltpu.SemaphoreType.DMA,
            pltpu.SemaphoreType.DMA,
        ],
        compiler_params=pltpu.CompilerParams(needs_layout_passes=False),
    )
    def gather_kernel(
        embT_hbm, idx_hbm, outT_hbm, row_v, idx_v, out_a, out_b, sem_row,
        sem_idx, sem_out_a, sem_out_b
    ):
        out_sems = (sem_out_a, sem_out_b)
        wid = lax.axis_index("s") * num_cores + lax.axis_index("c")
        idx_cp = pltpu.async_copy(idx_hbm, idx_v, sem_idx)
        bufs = (out_a, out_b)
        pending = [None, None]
        for rep in range(d_per_w):
            d = wid + rep * nw
            row_cp = pltpu.async_copy(embT_hbm.at[d], row_v, sem_row)
            if rep == 0:
                idx_cp.wait()
            row_cp.wait()
            for c in range(n_chunks):
                b = c % 2
                buf = bufs[b]
                if pending[b] is not None:
                    pending[b].wait()

                @plsc.parallel_loop(0, _CH // 16, unroll=8)
                def _(j, c=c, buf=buf):
                    iv = idx_v[pl.ds(c * _CH + j * 16, 16)]
                    buf[pl.ds(j * 16, 16)] = plsc.load_gather(row_v, [iv])

                pending[b] = pltpu.async_copy(
                    buf, outT_hbm.at[d, pl.ds(c * _CH, _CH)], out_sems[b]
                )
        for p in pending:
            if p is not None:
                p.wait()

    return gather_kernel


_gather = _make_gather()


def kernel(emb, token_index):
    outT = _gather(emb.T, token_index.astype(jnp.int32))
    return outT.T
